# baseline (speedup 1.0000x reference)
import functools

import jax
import jax.numpy as jnp
from jax import lax
from jax.experimental import pallas as pl
from jax.experimental.pallas import tpu as pltpu
from jax.experimental.pallas import tpu_sc as plsc


def kernel(x, table):
    B, S, D = x.shape
    NC, NS = 2, 16
    NW = NC * NS
    SPW = S // NW
    CS = 8
    NCH = SPW // CS
    CHUNK = CS * D

    xf = x.reshape(B * S * D)
    tf = table.reshape(-1)

    mesh = plsc.VectorSubcoreMesh(core_axis_name="c", subcore_axis_name="s")

    @functools.partial(
        pl.kernel,
        out_type=jax.ShapeDtypeStruct((B * S * D,), jnp.float32),
        mesh=mesh,
        scratch_types=[
            pltpu.VMEM((CHUNK,), jnp.float32),
            pltpu.VMEM((CHUNK,), jnp.float32),
        ],
    )
    def sc_add(x_hbm, t_hbm, o_hbm, t_v, xo_v):
        wid = lax.axis_index("s") * NC + lax.axis_index("c")
        s_base = wid * SPW

        @pl.loop(0, NCH)
        def _chunk(ci):
            s0 = s_base + ci * CS
            pltpu.sync_copy(t_hbm.at[pl.ds(s0 * D, CHUNK)], t_v)
            for b in range(B):
                off = (b * S + s0) * D
                pltpu.sync_copy(x_hbm.at[pl.ds(off, CHUNK)], xo_v)

                @pl.loop(0, CHUNK // 16, unroll=8)
                def _vec(i):
                    sl = pl.ds(i * 16, 16)
                    plsc.addupdate(xo_v.at[sl], t_v[sl])

                pltpu.sync_copy(xo_v, o_hbm.at[pl.ds(off, CHUNK)])

    out = sc_add(xf, tf)
    return out.reshape(B, S, D)
